# baseline (speedup 1.0000x reference)
import jax
import jax.numpy as jnp
from jax.experimental import pallas as pl
from jax.experimental.pallas import tpu as pltpu


def _tc_body(t_ref, h_ref, w_ref, x_ref, o_ref):
    xv = x_ref[0, 0]
    t = t_ref[0]
    t = t.reshape(1, 1, 256)
    h = h_ref[...].reshape(48, 1, 256)
    w = w_ref[...].reshape(1, 48, 256)
    o_ref[0, 0] = xv + (t + h) + w


def kernel(x, T_embed, H_embed, W_embed):
    B, T, H, W, C = x.shape
    h_emb = H_embed[:H]
    w_emb = W_embed[:W]
    t_emb = T_embed.reshape(T_embed.shape[0], 1, C)

    out = pl.pallas_call(
        _tc_body,
        grid=(B, T),
        in_specs=[
            pl.BlockSpec((1, 1, C), lambda b, t: (t, 0, 0)),
            pl.BlockSpec((H, C), lambda b, t: (0, 0)),
            pl.BlockSpec((W, C), lambda b, t: (0, 0)),
            pl.BlockSpec((1, 1, H, W, C), lambda b, t: (b, t, 0, 0, 0)),
        ],
        out_specs=pl.BlockSpec((1, 1, H, W, C), lambda b, t: (b, t, 0, 0, 0)),
        out_shape=jax.ShapeDtypeStruct(x.shape, x.dtype),
    )(t_emb, h_emb, w_emb, x)
    return out
